# Initial kernel scaffold; baseline (speedup 1.0000x reference)
import jax
import jax.numpy as jnp
from jax.experimental import pallas as pl


def kernel(x, edge_attr, params, edge_index, batch):
    raise NotImplementedError("write your pallas kernel here")


# speedup vs baseline: 1.7583x; 1.7583x over previous
import jax
import jax.numpy as jnp
from jax.experimental import pallas as pl

NN = 10000
EE = 160000
HID = 256
NG = 64
BN_EPS = 1e-5
NS_GAT = 0.2
NS = 0.01


def _gru(xg, h, w_ih, w_hh, b_ih, b_hh):
    gi = xg @ w_ih.T + b_ih
    gh = h @ w_hh.T + b_hh
    ir, iz, inn = jnp.split(gi, 3, axis=1)
    hr, hz, hn = jnp.split(gh, 3, axis=1)
    r = jax.nn.sigmoid(ir + hr)
    z = jax.nn.sigmoid(iz + hz)
    n = jnp.tanh(inn + r * hn)
    return (1.0 - z) * n + z * h


def _bn(h, gamma, beta):
    return h / jnp.sqrt(1.0 + BN_EPS) * gamma + beta


def _final_matmul_kernel(o_ref, w_ref, b_ref, out_ref):
    out_ref[...] = o_ref[...] @ w_ref[...] + b_ref[...]


def kernel(x, edge_attr, params, edge_index, batch):
    src, dst = edge_index[0], edge_index[1]
    x0 = x @ params['w_node'] + params['b_node']
    for lp in params['layers']:
        we = params['w_edge'] @ lp['lin_edge']
        be = params['b_edge'] @ lp['lin_edge']
        el = edge_attr @ we + be
        xl = x0 @ lp['lin_l']
        xr = x0 @ lp['lin_r']
        m = jax.nn.leaky_relu(xl[src] + xr[dst] + el, NS_GAT)
        logits = m @ lp['att']
        ex = jnp.exp(logits)
        denom = jax.ops.segment_sum(ex, dst, num_segments=NN)
        num = jax.ops.segment_sum(xl[src] * ex[:, None], dst, num_segments=NN)
        gat = jnp.where(denom[:, None] > 0, num / denom[:, None], 0.0) + lp['bias']
        h = jax.nn.elu(_bn(gat, lp['gamma'], lp['beta']))
        x0 = jax.nn.leaky_relu(
            _gru(h, x0, lp['w_ih'], lp['w_hh'], lp['b_ih'], lp['b_hh']), NS)

    out = jax.nn.leaky_relu(
        jax.ops.segment_sum(x0, batch, num_segments=NG), NS)
    mp = params['mol']
    xl_mol = x0 @ mp['lin_l']
    for _ in range(2):
        xr = out @ mp['lin_r']
        m = jax.nn.leaky_relu(xl_mol + xr[batch], NS_GAT)
        logits = m @ mp['att']
        ex = jnp.exp(logits)
        denom = jax.ops.segment_sum(ex, batch, num_segments=NG)
        num = jax.ops.segment_sum(xl_mol * ex[:, None], batch, num_segments=NG)
        gat = jnp.where(denom[:, None] > 0, num / denom[:, None], 0.0) + mp['bias']
        h = jax.nn.elu(_bn(gat, params['mol_gamma'], params['mol_beta']))
        out = jax.nn.leaky_relu(
            _gru(h, out, params['mol_w_ih'], params['mol_w_hh'],
                 params['mol_b_ih'], params['mol_b_hh']), NS)

    return pl.pallas_call(
        _final_matmul_kernel,
        out_shape=jax.ShapeDtypeStruct((NG, params['w_out'].shape[1]), jnp.float32),
    )(out, params['w_out'], params['b_out'])


# speedup vs baseline: 2.1383x; 1.2161x over previous
import jax
import jax.numpy as jnp
from jax import lax
from jax.experimental import pallas as pl
from jax.experimental.pallas import tpu as pltpu
from jax.experimental.pallas import tpu_sc as plsc

NN = 10000
EE = 160000
HID = 256
NG = 64
OUTD = 128
BN_EPS = 1e-5
NS_GAT = 0.2
NS = 0.01

NSC = 2
NSUB = 16
LANES = 16
NWORK = NSC * NSUB

EP = 163840
EPW1 = EP // NWORK
C1 = 64
NCH1 = EPW1 // C1
EPW2 = EP // NSUB
C2 = 64
NCH2 = EPW2 // C2
RW = 128
DEN_ROWS = ACC_ROWS_D = 80
ACC_ROWS = 10240
F32 = jnp.float32


def _lrelu(v, s):
    return jnp.maximum(v, v * s)


def _lane_shuffle(v, idx):
    dn = lax.GatherDimensionNumbers(offset_dims=(), collapsed_slice_dims=(0,),
                                    start_index_map=(0,))
    return lax.gather(v, idx[:, None], dn, slice_sizes=(1,),
                      mode=lax.GatherScatterMode.PROMISE_IN_BOUNDS)


def _x0_body(x_ref, w_ref, b_ref, o_ref):
    o_ref[...] = jnp.dot(x_ref[...], w_ref[...],
                         preferred_element_type=F32) + b_ref[...]


def _k_x0(x, w, b):
    return pl.pallas_call(
        _x0_body, grid=(10,),
        in_specs=[pl.BlockSpec((1000, 128), lambda i: (i, 0)),
                  pl.BlockSpec((128, HID), lambda i: (0, 0)),
                  pl.BlockSpec((1, HID), lambda i: (0, 0))],
        out_specs=pl.BlockSpec((1000, HID), lambda i: (i, 0)),
        out_shape=jax.ShapeDtypeStruct((NN, HID), F32),
    )(x, w, b)


def _el_body(ea_ref, w0, w1, w2, b0, b1, b2, o0, o1, o2):
    ea = ea_ref[...]
    o0[...] = jnp.dot(ea, w0[...], preferred_element_type=F32) + b0[...]
    o1[...] = jnp.dot(ea, w1[...], preferred_element_type=F32) + b1[...]
    o2[...] = jnp.dot(ea, w2[...], preferred_element_type=F32) + b2[...]


def _k_el(ea_p, ws, bs):
    wspec = pl.BlockSpec((16, HID), lambda i: (0, 0))
    bspec = pl.BlockSpec((1, HID), lambda i: (0, 0))
    espec = pl.BlockSpec((2048, HID), lambda i: (i, 0))
    return pl.pallas_call(
        _el_body, grid=(EP // 2048,),
        in_specs=[pl.BlockSpec((2048, 16), lambda i: (i, 0)),
                  wspec, wspec, wspec, bspec, bspec, bspec],
        out_specs=[espec, espec, espec],
        out_shape=[jax.ShapeDtypeStruct((EP, HID), F32)] * 3,
    )(ea_p, ws[0], ws[1], ws[2], bs[0], bs[1], bs[2])


def _pre_body(x_ref, wlo, whi, wr, oxl, oxr):
    x0 = x_ref[...]
    oxl[0, :, :] = jnp.dot(x0, wlo[...], preferred_element_type=F32)
    oxl[1, :, :] = jnp.dot(x0, whi[...], preferred_element_type=F32)
    oxr[...] = jnp.dot(x0, wr[...], preferred_element_type=F32)


def _k_pre(x0, lin_l_lo, lin_l_hi, lin_r):
    hspec = pl.BlockSpec((HID, 128), lambda i: (0, 0))
    return pl.pallas_call(
        _pre_body, grid=(10,),
        in_specs=[pl.BlockSpec((1000, HID), lambda i: (i, 0)),
                  hspec, hspec, pl.BlockSpec((HID, HID), lambda i: (0, 0))],
        out_specs=[pl.BlockSpec((2, 1000, 128), lambda i: (0, i, 0)),
                   pl.BlockSpec((1000, HID), lambda i: (i, 0))],
        out_shape=[jax.ShapeDtypeStruct((2, NN, 128), F32),
                   jax.ShapeDtypeStruct((NN, HID), F32)],
    )(x0, lin_l_lo, lin_l_hi, lin_r)


def _post_body(alo, ahi, den_ref, x_ref, bias, gamma, beta,
               wih, whh, bih, bhh, o_ref):
    num = jnp.concatenate([alo[...], ahi[...]], axis=1)
    den = den_ref[...]
    gat = jnp.where(den > 0, num / den, 0.0) + bias[...]
    h = gat * gamma[...] + beta[...]
    h = jnp.where(h > 0, h, jnp.exp(h) - 1.0)
    xo = x_ref[...]
    gi = jnp.dot(h, wih[...], preferred_element_type=F32) + bih[...]
    gh = jnp.dot(xo, whh[...], preferred_element_type=F32) + bhh[...]
    r = jax.nn.sigmoid(gi[:, :HID] + gh[:, :HID])
    z = jax.nn.sigmoid(gi[:, HID:2 * HID] + gh[:, HID:2 * HID])
    n = jnp.tanh(gi[:, 2 * HID:] + r * gh[:, 2 * HID:])
    o_ref[...] = _lrelu((1.0 - z) * n + z * xo, NS)


def _k_post(acc_lo, acc_hi, den_col, x0, bias, gammas, beta,
            wihT, whhT, bih, bhh):
    vspec = pl.BlockSpec((1, HID), lambda i: (0, 0))
    gspec = pl.BlockSpec((1, 3 * HID), lambda i: (0, 0))
    return pl.pallas_call(
        _post_body, grid=(10,),
        in_specs=[pl.BlockSpec((1000, 128), lambda i: (i, 0)),
                  pl.BlockSpec((1000, 128), lambda i: (i, 0)),
                  pl.BlockSpec((1000, 1), lambda i: (i, 0)),
                  pl.BlockSpec((1000, HID), lambda i: (i, 0)),
                  vspec, vspec, vspec,
                  pl.BlockSpec((HID, 3 * HID), lambda i: (0, 0)),
                  pl.BlockSpec((HID, 3 * HID), lambda i: (0, 0)),
                  gspec, gspec],
        out_specs=pl.BlockSpec((1000, HID), lambda i: (i, 0)),
        out_shape=jax.ShapeDtypeStruct((NN, HID), F32),
    )(acc_lo, acc_hi, den_col, x0, bias, gammas, beta, wihT, whhT, bih, bhh)


def _molpre_body(x_ref, w_ref, bat_ref, oxl, oout):
    x3 = x_ref[...]
    oxl[...] = jnp.dot(x3, w_ref[...], preferred_element_type=F32)
    gid = lax.broadcasted_iota(jnp.int32, (NG, NN), 0)
    oh = (gid == jnp.reshape(bat_ref[...], (1, NN))).astype(F32)
    oout[...] = _lrelu(jnp.dot(oh, x3, preferred_element_type=F32), NS)


def _k_molpre(x3, lin_l, batf):
    return pl.pallas_call(
        _molpre_body,
        out_shape=[jax.ShapeDtypeStruct((NN, HID), F32),
                   jax.ShapeDtypeStruct((NG, HID), F32)],
    )(x3, lin_l, batf)


def _mol_body(xl_ref, bat_ref, op_ref, wr, att, bias, gamma, beta,
              wih, whh, bih, bhh, o_ref):
    xl = xl_ref[...]
    outp = op_ref[...]
    xr = jnp.dot(outp, wr[...], preferred_element_type=F32)
    batf = bat_ref[...]
    ohT = (batf == lax.broadcasted_iota(jnp.int32, (NN, NG), 1)).astype(F32)
    xr_exp = jnp.dot(ohT, xr, preferred_element_type=F32)
    m = _lrelu(xl + xr_exp, NS_GAT)
    ex = jnp.exp(jnp.dot(m, att[...], preferred_element_type=F32))
    oh = (lax.broadcasted_iota(jnp.int32, (NG, NN), 0)
          == jnp.reshape(batf, (1, NN))).astype(F32)
    num = jnp.dot(oh, xl * ex, preferred_element_type=F32)
    den = jnp.dot(oh, ex, preferred_element_type=F32)
    gat = jnp.where(den > 0, num / den, 0.0) + bias[...]
    h = gat * gamma[...] + beta[...]
    h = jnp.where(h > 0, h, jnp.exp(h) - 1.0)
    gi = jnp.dot(h, wih[...], preferred_element_type=F32) + bih[...]
    gh = jnp.dot(outp, whh[...], preferred_element_type=F32) + bhh[...]
    r = jax.nn.sigmoid(gi[:, :HID] + gh[:, :HID])
    z = jax.nn.sigmoid(gi[:, HID:2 * HID] + gh[:, HID:2 * HID])
    n = jnp.tanh(gi[:, 2 * HID:] + r * gh[:, 2 * HID:])
    o_ref[...] = _lrelu((1.0 - z) * n + z * outp, NS)


def _k_mol(xl_mol, batf, outp, wr, att, bias, gamma, beta, wih, whh, bih, bhh):
    return pl.pallas_call(
        _mol_body,
        out_shape=jax.ShapeDtypeStruct((NG, HID), F32),
    )(xl_mol, batf, outp, wr, att, bias, gamma, beta, wih, whh, bih, bhh)


def _final_body(o_ref, w_ref, b_ref, out_ref):
    out_ref[...] = jnp.dot(o_ref[...], w_ref[...],
                           preferred_element_type=F32) + b_ref[...]


def _k_final(out, w, b):
    return pl.pallas_call(
        _final_body,
        out_shape=jax.ShapeDtypeStruct((NG, OUTD), F32),
    )(out, w, b)


_MESH = plsc.VectorSubcoreMesh(core_axis_name="c", subcore_axis_name="s",
                               num_cores=NSC, num_subcores=NSUB)


_SC_PARAMS = pltpu.CompilerParams(needs_layout_passes=False)


def _sc_pass1_body(xl_cat, xr, el, srcp, dstg, atts, ex_out,
                   att_v, src_b, srch_b, dst_b, xlo_b, xhi_b, xr_b, el_b,
                   ex_b, sem):
    c = lax.axis_index("c")
    s = lax.axis_index("s")
    wid = s * NSC + c
    pltpu.sync_copy(atts, att_v)
    iota16 = lax.broadcasted_iota(jnp.int32, (LANES,), 0)

    def chunk_body(ch, carry):
        base = wid * EPW1 + ch * C1
        pltpu.sync_copy(srcp.at[pl.ds(base, C1)], src_b)
        pltpu.sync_copy(dstg.at[pl.ds(base, C1)], dst_b)
        for g in range(C1 // LANES):
            sl = pl.ds(g * LANES, LANES)
            srch_b[sl] = src_b[sl] + NN
        d1 = pltpu.async_copy(xl_cat.at[src_b], xlo_b, sem)
        d2 = pltpu.async_copy(xl_cat.at[srch_b], xhi_b, sem)
        d3 = pltpu.async_copy(xr.at[dst_b], xr_b, sem)
        pltpu.sync_copy(el.at[pl.ds(base, C1)], el_b)
        d1.wait()
        d2.wait()
        d3.wait()

        def edge_body(i, carry2):
            acc = jnp.zeros((LANES,), F32)
            for k in range(16):
                sl = pl.ds(k * LANES, LANES)
                if k < 8:
                    xlv = xlo_b[i, sl]
                else:
                    xlv = xhi_b[i, pl.ds((k - 8) * LANES, LANES)]
                m = xlv + xr_b[i, sl] + el_b[i, sl]
                m = jnp.maximum(m, m * NS_GAT)
                acc = acc + m * att_v[sl]
            for sh in (1, 2, 4, 8):
                acc = acc + _lane_shuffle(acc, iota16 ^ sh)
            ex_b[i, :] = jnp.exp(acc)
            return carry2

        lax.fori_loop(0, C1, edge_body, 0)
        pltpu.sync_copy(ex_b, ex_out.at[pl.ds(base, C1)])
        return carry

    lax.fori_loop(0, NCH1, chunk_body, 0)


def _sc_pass1(xl_cat, xr, el, src_p, dstg_p, att):
    return pl.kernel(
        _sc_pass1_body,
        out_type=jax.ShapeDtypeStruct((EP, LANES), F32),
        mesh=_MESH,
        compiler_params=_SC_PARAMS,
        scratch_types=[
            pltpu.VMEM((HID,), F32),
            pltpu.VMEM((C1,), jnp.int32),
            pltpu.VMEM((C1,), jnp.int32),
            pltpu.VMEM((C1,), jnp.int32),
            pltpu.VMEM((C1, 128), F32),
            pltpu.VMEM((C1, 128), F32),
            pltpu.VMEM((C1, HID), F32),
            pltpu.VMEM((C1, HID), F32),
            pltpu.VMEM((C1, LANES), F32),
            pltpu.SemaphoreType.DMA,
        ],
    )(xl_cat, xr, el, src_p, dstg_p, att)


def _sc_pass2_body(xl_cat, srcp, dsts, ex, accf_out, den_out,
                   src_b, dst_b, drow_b, ex_b, xl_b, contrib, contrib_d,
                   spacc_f, spacc_d, sem):
    c = lax.axis_index("c")
    s = lax.axis_index("s")
    iota16 = lax.broadcasted_iota(jnp.int32, (LANES,), 0)
    zero16 = jnp.zeros((LANES,), F32)
    ngrp = C2 // LANES

    def zrow(r, carry):
        for kk in range(128 // LANES):
            sl = pl.ds(kk * LANES, LANES)
            contrib[r, sl] = zero16
            contrib_d[r, sl] = zero16
        return carry
    lax.fori_loop(0, C2, zrow, 0)

    def zacc(z, carry):
        pltpu.sync_copy(contrib, spacc_f.at[pl.ds(s * 640 + z * C2, C2)])
        return carry
    lax.fori_loop(0, ACC_ROWS // NSUB // C2, zacc, 0)

    @pl.when(s == 0)
    def _():
        pltpu.sync_copy(contrib, spacc_d.at[pl.ds(0, C2)])
        pltpu.sync_copy(contrib.at[pl.ds(0, DEN_ROWS - C2)],
                        spacc_d.at[pl.ds(C2, DEN_ROWS - C2)])
    plsc.subcore_barrier()

    def chunk_body(ch, carry):
        base = s * EPW2 + ch * C2
        pltpu.sync_copy(srcp.at[pl.ds(base, C2)], src_b)
        pltpu.sync_copy(dsts.at[pl.ds(base, C2)], dst_b)
        pltpu.sync_copy(ex.at[pl.ds(base, C2)], ex_b)
        for g in range(ngrp):
            sl = pl.ds(g * LANES, LANES)
            src_b[sl] = src_b[sl] + c * NN
        pltpu.async_copy(xl_cat.at[src_b], xl_b, sem).wait()

        def edge_body(i, cr):
            exv = ex_b[i, :]
            for k in range(8):
                sl = pl.ds(k * LANES, LANES)
                contrib[i, sl] = xl_b[i, sl] * exv
            return cr
        lax.fori_loop(0, C2, edge_body, 0)

        for g in range(ngrp):
            sl = pl.ds(g * LANES, LANES)
            rowsg = g * LANES + iota16
            dstv = dst_b[sl]
            drow_b[sl] = jnp.right_shift(dstv, 7)
            exg = plsc.load_gather(ex_b, [rowsg, jnp.zeros((LANES,),
                                                           jnp.int32)])
            plsc.store_scatter(contrib_d, [rowsg, dstv & 127], exg)

        pltpu.sync_copy(contrib, spacc_f.at[dst_b], add=True)
        pltpu.sync_copy(contrib_d, spacc_d.at[drow_b], add=True)

        for g in range(ngrp):
            sl = pl.ds(g * LANES, LANES)
            rowsg = g * LANES + iota16
            dstv = dst_b[sl]
            plsc.store_scatter(contrib_d, [rowsg, dstv & 127], zero16)
        return carry

    lax.fori_loop(0, NCH2, chunk_body, 0)
    plsc.subcore_barrier()

    pltpu.sync_copy(spacc_f.at[pl.ds(s * 640, 640)],
                    accf_out.at[pl.ds(c * ACC_ROWS + s * 640, 640)])

    @pl.when(s == 0)
    def _():
        pltpu.sync_copy(spacc_d, den_out.at[pl.ds(c * DEN_ROWS, DEN_ROWS)])


def _sc_pass2(xl_cat, src_p, dsts_p, ex):
    return pl.kernel(
        _sc_pass2_body,
        out_type=(jax.ShapeDtypeStruct((2 * ACC_ROWS, 128), F32),
                  jax.ShapeDtypeStruct((2 * DEN_ROWS, 128), F32)),
        mesh=_MESH,
        compiler_params=_SC_PARAMS,
        scratch_types=[
            pltpu.VMEM((C2,), jnp.int32),
            pltpu.VMEM((C2,), jnp.int32),
            pltpu.VMEM((C2,), jnp.int32),
            pltpu.VMEM((C2, LANES), F32),
            pltpu.VMEM((C2, 128), F32),
            pltpu.VMEM((C2, 128), F32),
            pltpu.VMEM((C2, 128), F32),
            pltpu.VMEM_SHARED((ACC_ROWS, 128), F32),
            pltpu.VMEM_SHARED((DEN_ROWS, 128), F32),
            pltpu.SemaphoreType.DMA,
        ],
    )(xl_cat, src_p, dsts_p, ex)


def kernel(x, edge_attr, params, edge_index, batch):
    src, dst = edge_index[0], edge_index[1]
    pad = EP - EE
    src_p = jnp.pad(src, (0, pad))
    dstg_p = jnp.pad(dst, (0, pad))
    dsts_p = jnp.pad(dst, (0, pad), constant_values=NN)
    ea_p = jnp.pad(edge_attr, ((0, pad), (0, 0)))
    batf = batch.reshape(NN, 1)

    we = [params['w_edge'] @ lp['lin_edge'] for lp in params['layers']]
    be = [(params['b_edge'] @ lp['lin_edge']).reshape(1, HID)
          for lp in params['layers']]

    x0 = _k_x0(x, params['w_node'], params['b_node'].reshape(1, HID))
    els = _k_el(ea_p, we, be)

    for li, lp in enumerate(params['layers']):
        xl2, xr = _k_pre(x0, lp['lin_l'][:, :128],
                         lp['lin_l'][:, 128:], lp['lin_r'])
        xl_cat = xl2.reshape(2 * NN, 128)
        ex = _sc_pass1(xl_cat, xr, els[li], src_p, dstg_p, lp['att'])
        acc_f, den = _sc_pass2(xl_cat, src_p, dsts_p, ex)
        den_col = den[:DEN_ROWS].reshape(DEN_ROWS * 128)[:NN].reshape(NN, 1)
        gscale = (lp['gamma'] / jnp.sqrt(1.0 + BN_EPS)).reshape(1, HID)
        x0 = _k_post(acc_f[:NN], acc_f[ACC_ROWS:ACC_ROWS + NN], den_col, x0,
                     lp['bias'].reshape(1, HID), gscale,
                     lp['beta'].reshape(1, HID),
                     lp['w_ih'].T, lp['w_hh'].T,
                     lp['b_ih'].reshape(1, 3 * HID),
                     lp['b_hh'].reshape(1, 3 * HID))

    mp = params['mol']
    xl_mol, out = _k_molpre(x0, mp['lin_l'], batf)
    mol_gscale = (params['mol_gamma'] / jnp.sqrt(1.0 + BN_EPS)).reshape(1, HID)
    for _ in range(2):
        out = _k_mol(xl_mol, batf, out, mp['lin_r'], mp['att'].reshape(HID, 1),
                     mp['bias'].reshape(1, HID), mol_gscale,
                     params['mol_beta'].reshape(1, HID),
                     params['mol_w_ih'].T, params['mol_w_hh'].T,
                     params['mol_b_ih'].reshape(1, 3 * HID),
                     params['mol_b_hh'].reshape(1, 3 * HID))
    return _k_final(out, params['w_out'], params['b_out'].reshape(1, OUTD))


# speedup vs baseline: 3.2285x; 1.5098x over previous
import jax
import jax.numpy as jnp
from jax import lax
from jax.experimental import pallas as pl
from jax.experimental.pallas import tpu as pltpu
from jax.experimental.pallas import tpu_sc as plsc

NN = 10000
EE = 160000
HID = 256
NG = 64
OUTD = 128
BN_EPS = 1e-5
NS_GAT = 0.2
NS = 0.01

NSC = 2
NSUB = 16
LANES = 16
NWORK = NSC * NSUB

EP = 163840
EPW1 = EP // NWORK
C1 = 64
NCH1 = EPW1 // C1
EPW2 = EP // NSUB
C2 = 64
NCH2 = EPW2 // C2
RW = 128
DEN_ROWS = 80
DEN_BASE = 10016
ACC_ROWS = 10112
F32 = jnp.float32


def _lrelu(v, s):
    return jnp.maximum(v, v * s)


def _lane_shuffle(v, idx):
    dn = lax.GatherDimensionNumbers(offset_dims=(), collapsed_slice_dims=(0,),
                                    start_index_map=(0,))
    return lax.gather(v, idx[:, None], dn, slice_sizes=(1,),
                      mode=lax.GatherScatterMode.PROMISE_IN_BOUNDS)


def _x0_body(x_ref, w_ref, b_ref, o_ref):
    o_ref[...] = jnp.dot(x_ref[...], w_ref[...],
                         preferred_element_type=F32) + b_ref[...]


def _k_x0(x, w, b):
    return pl.pallas_call(
        _x0_body, grid=(10,),
        in_specs=[pl.BlockSpec((1000, 128), lambda i: (i, 0)),
                  pl.BlockSpec((128, HID), lambda i: (0, 0)),
                  pl.BlockSpec((1, HID), lambda i: (0, 0))],
        out_specs=pl.BlockSpec((1000, HID), lambda i: (i, 0)),
        out_shape=jax.ShapeDtypeStruct((NN, HID), F32),
    )(x, w, b)


def _el_body(ea_ref, w0, w1, w2, b0, b1, b2, o0, o1, o2):
    ea = ea_ref[...]
    o0[...] = jnp.dot(ea, w0[...], preferred_element_type=F32) + b0[...]
    o1[...] = jnp.dot(ea, w1[...], preferred_element_type=F32) + b1[...]
    o2[...] = jnp.dot(ea, w2[...], preferred_element_type=F32) + b2[...]


def _k_el(ea_p, ws, bs):
    wspec = pl.BlockSpec((16, HID), lambda i: (0, 0))
    bspec = pl.BlockSpec((1, HID), lambda i: (0, 0))
    espec = pl.BlockSpec((2048, HID), lambda i: (i, 0))
    return pl.pallas_call(
        _el_body, grid=(EP // 2048,),
        in_specs=[pl.BlockSpec((2048, 16), lambda i: (i, 0)),
                  wspec, wspec, wspec, bspec, bspec, bspec],
        out_specs=[espec, espec, espec],
        out_shape=[jax.ShapeDtypeStruct((EP, HID), F32)] * 3,
    )(ea_p, ws[0], ws[1], ws[2], bs[0], bs[1], bs[2])


def _pre_body(x_ref, wlo, whi, wr, oxl, oxr):
    x0 = x_ref[...]
    oxl[0, :, :] = jnp.dot(x0, wlo[...], preferred_element_type=F32)
    oxl[1, :, :] = jnp.dot(x0, whi[...], preferred_element_type=F32)
    oxr[...] = jnp.dot(x0, wr[...], preferred_element_type=F32)


def _k_pre(x0, lin_l_lo, lin_l_hi, lin_r):
    hspec = pl.BlockSpec((HID, 128), lambda i: (0, 0))
    return pl.pallas_call(
        _pre_body, grid=(10,),
        in_specs=[pl.BlockSpec((1000, HID), lambda i: (i, 0)),
                  hspec, hspec, pl.BlockSpec((HID, HID), lambda i: (0, 0))],
        out_specs=[pl.BlockSpec((2, 1000, 128), lambda i: (0, i, 0)),
                   pl.BlockSpec((1000, HID), lambda i: (i, 0))],
        out_shape=[jax.ShapeDtypeStruct((2, NN, 128), F32),
                   jax.ShapeDtypeStruct((NN, HID), F32)],
    )(x0, lin_l_lo, lin_l_hi, lin_r)


def _post_body(alo, ahi, den_ref, x_ref, bias, gamma, beta,
               wih, whh, bih, bhh, o_ref):
    num = jnp.concatenate([alo[...], ahi[...]], axis=1)
    den = den_ref[...]
    gat = jnp.where(den > 0, num / den, 0.0) + bias[...]
    h = gat * gamma[...] + beta[...]
    h = jnp.where(h > 0, h, jnp.exp(h) - 1.0)
    xo = x_ref[...]
    gi = jnp.dot(h, wih[...], preferred_element_type=F32) + bih[...]
    gh = jnp.dot(xo, whh[...], preferred_element_type=F32) + bhh[...]
    r = jax.nn.sigmoid(gi[:, :HID] + gh[:, :HID])
    z = jax.nn.sigmoid(gi[:, HID:2 * HID] + gh[:, HID:2 * HID])
    n = jnp.tanh(gi[:, 2 * HID:] + r * gh[:, 2 * HID:])
    o_ref[...] = _lrelu((1.0 - z) * n + z * xo, NS)


def _k_post(acc_lo, acc_hi, den_col, x0, bias, gammas, beta,
            wihT, whhT, bih, bhh):
    vspec = pl.BlockSpec((1, HID), lambda i: (0, 0))
    gspec = pl.BlockSpec((1, 3 * HID), lambda i: (0, 0))
    return pl.pallas_call(
        _post_body, grid=(10,),
        in_specs=[pl.BlockSpec((1000, 128), lambda i: (i, 0)),
                  pl.BlockSpec((1000, 128), lambda i: (i, 0)),
                  pl.BlockSpec((1000, 1), lambda i: (i, 0)),
                  pl.BlockSpec((1000, HID), lambda i: (i, 0)),
                  vspec, vspec, vspec,
                  pl.BlockSpec((HID, 3 * HID), lambda i: (0, 0)),
                  pl.BlockSpec((HID, 3 * HID), lambda i: (0, 0)),
                  gspec, gspec],
        out_specs=pl.BlockSpec((1000, HID), lambda i: (i, 0)),
        out_shape=jax.ShapeDtypeStruct((NN, HID), F32),
    )(acc_lo, acc_hi, den_col, x0, bias, gammas, beta, wihT, whhT, bih, bhh)


def _molpre_body(x_ref, w_ref, bat_ref, oxl, oout):
    x3 = x_ref[...]
    oxl[...] = jnp.dot(x3, w_ref[...], preferred_element_type=F32)
    gid = lax.broadcasted_iota(jnp.int32, (NG, NN), 0)
    oh = (gid == jnp.reshape(bat_ref[...], (1, NN))).astype(F32)
    oout[...] = _lrelu(jnp.dot(oh, x3, preferred_element_type=F32), NS)


def _k_molpre(x3, lin_l, batf):
    return pl.pallas_call(
        _molpre_body,
        out_shape=[jax.ShapeDtypeStruct((NN, HID), F32),
                   jax.ShapeDtypeStruct((NG, HID), F32)],
    )(x3, lin_l, batf)


def _mol_body(xl_ref, bat_ref, op_ref, wr, att, bias, gamma, beta,
              wih, whh, bih, bhh, o_ref):
    xl = xl_ref[...]
    outp = op_ref[...]
    xr = jnp.dot(outp, wr[...], preferred_element_type=F32)
    batf = bat_ref[...]
    ohT = (batf == lax.broadcasted_iota(jnp.int32, (NN, NG), 1)).astype(F32)
    xr_exp = jnp.dot(ohT, xr, preferred_element_type=F32)
    m = _lrelu(xl + xr_exp, NS_GAT)
    ex = jnp.exp(jnp.dot(m, att[...], preferred_element_type=F32))
    oh = (lax.broadcasted_iota(jnp.int32, (NG, NN), 0)
          == jnp.reshape(batf, (1, NN))).astype(F32)
    num = jnp.dot(oh, xl * ex, preferred_element_type=F32)
    den = jnp.dot(oh, ex, preferred_element_type=F32)
    gat = jnp.where(den > 0, num / den, 0.0) + bias[...]
    h = gat * gamma[...] + beta[...]
    h = jnp.where(h > 0, h, jnp.exp(h) - 1.0)
    gi = jnp.dot(h, wih[...], preferred_element_type=F32) + bih[...]
    gh = jnp.dot(outp, whh[...], preferred_element_type=F32) + bhh[...]
    r = jax.nn.sigmoid(gi[:, :HID] + gh[:, :HID])
    z = jax.nn.sigmoid(gi[:, HID:2 * HID] + gh[:, HID:2 * HID])
    n = jnp.tanh(gi[:, 2 * HID:] + r * gh[:, 2 * HID:])
    o_ref[...] = _lrelu((1.0 - z) * n + z * outp, NS)


def _k_mol(xl_mol, batf, outp, wr, att, bias, gamma, beta, wih, whh, bih, bhh):
    return pl.pallas_call(
        _mol_body,
        out_shape=jax.ShapeDtypeStruct((NG, HID), F32),
    )(xl_mol, batf, outp, wr, att, bias, gamma, beta, wih, whh, bih, bhh)


def _final_body(o_ref, w_ref, b_ref, out_ref):
    out_ref[...] = jnp.dot(o_ref[...], w_ref[...],
                           preferred_element_type=F32) + b_ref[...]


def _k_final(out, w, b):
    return pl.pallas_call(
        _final_body,
        out_shape=jax.ShapeDtypeStruct((NG, OUTD), F32),
    )(out, w, b)


_MESH = plsc.VectorSubcoreMesh(core_axis_name="c", subcore_axis_name="s",
                               num_cores=NSC, num_subcores=NSUB)


_SC_PARAMS = pltpu.CompilerParams(needs_layout_passes=False)


def _sc_pass1_body(xl_cat, xr, el, srcp, dstg, atts, ex_out,
                   att_v, src_b, srch_b, dst_b, xlo_b, xhi_b, xr_b, el_b,
                   ex_b, sem0, sem1):
    c = lax.axis_index("c")
    s = lax.axis_index("s")
    wid = s * NSC + c
    pltpu.sync_copy(atts, att_v)
    iota16 = lax.broadcasted_iota(jnp.int32, (LANES,), 0)
    sems = (sem0, sem1)
    att_regs = [att_v[pl.ds(k * LANES, LANES)] for k in range(16)]

    def issue(g, b):
        base = wid * EPW1 + g * C1
        pltpu.sync_copy(srcp.at[pl.ds(base, C1)], src_b.at[b])
        pltpu.sync_copy(dstg.at[pl.ds(base, C1)], dst_b.at[b])
        for gg in range(C1 // LANES):
            sl = pl.ds(gg * LANES, LANES)
            srch_b[b, sl] = src_b[b, sl] + NN
        pltpu.async_copy(xl_cat.at[src_b.at[b]], xlo_b.at[b], sems[b])
        pltpu.async_copy(xl_cat.at[srch_b.at[b]], xhi_b.at[b], sems[b])
        pltpu.async_copy(xr.at[dst_b.at[b]], xr_b.at[b], sems[b])
        pltpu.async_copy(el.at[pl.ds(base, C1)], el_b.at[b], sems[b])

    def drain(b):
        pltpu.make_async_copy(xl_cat.at[src_b.at[b]], xlo_b.at[b],
                              sems[b]).wait()
        pltpu.make_async_copy(xl_cat.at[srch_b.at[b]], xhi_b.at[b],
                              sems[b]).wait()
        pltpu.make_async_copy(xr.at[dst_b.at[b]], xr_b.at[b], sems[b]).wait()
        pltpu.make_async_copy(el.at[pl.ds(0, C1)], el_b.at[b], sems[b]).wait()

    def compute(g, b):
        base = wid * EPW1 + g * C1

        def edge_body(i, carry2):
            acc = jnp.zeros((LANES,), F32)
            for k in range(16):
                sl = pl.ds(k * LANES, LANES)
                if k < 8:
                    xlv = xlo_b[b, i, sl]
                else:
                    xlv = xhi_b[b, i, pl.ds((k - 8) * LANES, LANES)]
                m = xlv + xr_b[b, i, sl] + el_b[b, i, sl]
                m = jnp.maximum(m, m * NS_GAT)
                acc = acc + m * att_regs[k]
            for sh in (1, 2, 4, 8):
                acc = acc + _lane_shuffle(acc, iota16 ^ sh)
            ex_b[i, :] = jnp.exp(acc)
            return carry2

        lax.fori_loop(0, C1, edge_body, 0)
        pltpu.sync_copy(ex_b, ex_out.at[pl.ds(base, C1)])

    issue(0, 0)
    issue(1, 1)

    def pair_body(p, carry):
        for b in (0, 1):
            g = 2 * p + b
            drain(b)
            compute(g, b)
            issue(g + 2, b)
        return carry

    lax.fori_loop(0, (NCH1 - 2) // 2, pair_body, 0)
    for b in (0, 1):
        drain(b)
        compute(NCH1 - 2 + b, b)


def _sc_pass1(xl_cat, xr, el, src_p, dstg_p, att):
    return pl.kernel(
        _sc_pass1_body,
        out_type=jax.ShapeDtypeStruct((EP, LANES), F32),
        mesh=_MESH,
        compiler_params=_SC_PARAMS,
        scratch_types=[
            pltpu.VMEM((HID,), F32),
            pltpu.VMEM((2, C1), jnp.int32),
            pltpu.VMEM((2, C1), jnp.int32),
            pltpu.VMEM((2, C1), jnp.int32),
            pltpu.VMEM((2, C1, 128), F32),
            pltpu.VMEM((2, C1, 128), F32),
            pltpu.VMEM((2, C1, HID), F32),
            pltpu.VMEM((2, C1, HID), F32),
            pltpu.VMEM((C1, LANES), F32),
            pltpu.SemaphoreType.DMA,
            pltpu.SemaphoreType.DMA,
        ],
    )(xl_cat, xr, el, src_p, dstg_p, att)


def _sc_pass2_body(xl_cat, srcp, dsts, ex, accf_out, den_out,
                   src_b, dst_b, drow_b, ex_b, xl_b, contrib, contrib_d,
                   spacc_f, sem0, sem1):
    c = lax.axis_index("c")
    s = lax.axis_index("s")
    iota16 = lax.broadcasted_iota(jnp.int32, (LANES,), 0)
    zero16 = jnp.zeros((LANES,), F32)
    ngrp = C2 // LANES
    sems = (sem0, sem1)

    def zrow(r, carry):
        for kk in range(128 // LANES):
            sl = pl.ds(kk * LANES, LANES)
            contrib[r, sl] = zero16
            contrib_d[r, sl] = zero16
        return carry
    lax.fori_loop(0, C2, zrow, 0)

    def zacc(z, carry):
        pltpu.sync_copy(contrib, spacc_f.at[pl.ds(s * 632 + z * C2, C2)])
        return carry
    lax.fori_loop(0, 9, zacc, 0)
    pltpu.sync_copy(contrib.at[pl.ds(0, 56)],
                    spacc_f.at[pl.ds(s * 632 + 576, 56)])

    plsc.subcore_barrier()

    def issue(g, b):
        base = s * EPW2 + g * C2
        pltpu.sync_copy(srcp.at[pl.ds(base, C2)], src_b.at[b])
        pltpu.sync_copy(dsts.at[pl.ds(base, C2)], dst_b.at[b])
        pltpu.sync_copy(ex.at[pl.ds(base, C2)], ex_b.at[b])
        for g2 in range(ngrp):
            sl = pl.ds(g2 * LANES, LANES)
            src_b[b, sl] = src_b[b, sl] + c * NN
        pltpu.async_copy(xl_cat.at[src_b.at[b]], xl_b.at[b], sems[b])

    def drain(b):
        pltpu.make_async_copy(xl_cat.at[src_b.at[b]], xl_b.at[b],
                              sems[b]).wait()

    def compute(g, b):
        def edge_body(i, cr):
            exv = ex_b[b, i, :]
            for k in range(8):
                sl = pl.ds(k * LANES, LANES)
                contrib[i, sl] = xl_b[b, i, sl] * exv
            return cr
        lax.fori_loop(0, C2, edge_body, 0)

        for g2 in range(ngrp):
            sl = pl.ds(g2 * LANES, LANES)
            rowsg = g2 * LANES + iota16
            dstv = dst_b[b, sl]
            drow_b[sl] = DEN_BASE + jnp.right_shift(dstv, 7)
            exg = plsc.load_gather(
                ex_b, [jnp.full((LANES,), b, jnp.int32), rowsg,
                       jnp.zeros((LANES,), jnp.int32)])
            plsc.store_scatter(contrib_d, [rowsg, dstv & 127], exg)

        pltpu.sync_copy(contrib, spacc_f.at[dst_b.at[b]], add=True)
        pltpu.sync_copy(contrib_d, spacc_f.at[drow_b], add=True)

        for g2 in range(ngrp):
            sl = pl.ds(g2 * LANES, LANES)
            rowsg = g2 * LANES + iota16
            dstv = dst_b[b, sl]
            plsc.store_scatter(contrib_d, [rowsg, dstv & 127], zero16)

    issue(0, 0)
    issue(1, 1)

    def pair_body(p, carry):
        for b in (0, 1):
            g = 2 * p + b
            drain(b)
            compute(g, b)
            issue(g + 2, b)
        return carry

    lax.fori_loop(0, (NCH2 - 2) // 2, pair_body, 0)
    for b in (0, 1):
        drain(b)
        compute(NCH2 - 2 + b, b)

    plsc.subcore_barrier()

    pltpu.sync_copy(spacc_f.at[pl.ds(s * 632, 632)],
                    accf_out.at[pl.ds(c * ACC_ROWS + s * 632, 632)])

    @pl.when(s == 0)
    def _():
        pltpu.sync_copy(spacc_f.at[pl.ds(DEN_BASE, DEN_ROWS)],
                        den_out.at[pl.ds(c * DEN_ROWS, DEN_ROWS)])


def _sc_pass2(xl_cat, src_p, dsts_p, ex):
    return pl.kernel(
        _sc_pass2_body,
        out_type=(jax.ShapeDtypeStruct((2 * ACC_ROWS, 128), F32),
                  jax.ShapeDtypeStruct((2 * DEN_ROWS, 128), F32)),
        mesh=_MESH,
        compiler_params=_SC_PARAMS,
        scratch_types=[
            pltpu.VMEM((2, C2), jnp.int32),
            pltpu.VMEM((2, C2), jnp.int32),
            pltpu.VMEM((C2,), jnp.int32),
            pltpu.VMEM((2, C2, LANES), F32),
            pltpu.VMEM((2, C2, 128), F32),
            pltpu.VMEM((C2, 128), F32),
            pltpu.VMEM((C2, 128), F32),
            pltpu.VMEM_SHARED((ACC_ROWS, 128), F32),
            pltpu.SemaphoreType.DMA,
            pltpu.SemaphoreType.DMA,
        ],
    )(xl_cat, src_p, dsts_p, ex)


def kernel(x, edge_attr, params, edge_index, batch):
    src, dst = edge_index[0], edge_index[1]
    pad = EP - EE
    src_p = jnp.pad(src, (0, pad))
    dstg_p = jnp.pad(dst, (0, pad))
    dsts_p = jnp.pad(dst, (0, pad), constant_values=NN)
    ea_p = jnp.pad(edge_attr, ((0, pad), (0, 0)))
    batf = batch.reshape(NN, 1)

    we = [params['w_edge'] @ lp['lin_edge'] for lp in params['layers']]
    be = [(params['b_edge'] @ lp['lin_edge']).reshape(1, HID)
          for lp in params['layers']]

    x0 = _k_x0(x, params['w_node'], params['b_node'].reshape(1, HID))
    els = _k_el(ea_p, we, be)

    for li, lp in enumerate(params['layers']):
        xl2, xr = _k_pre(x0, lp['lin_l'][:, :128],
                         lp['lin_l'][:, 128:], lp['lin_r'])
        xl_cat = xl2.reshape(2 * NN, 128)
        ex = _sc_pass1(xl_cat, xr, els[li], src_p, dstg_p, lp['att'])
        acc_f, den = _sc_pass2(xl_cat, src_p, dsts_p, ex)
        den_col = den[:DEN_ROWS].reshape(DEN_ROWS * 128)[:NN].reshape(NN, 1)
        gscale = (lp['gamma'] / jnp.sqrt(1.0 + BN_EPS)).reshape(1, HID)
        x0 = _k_post(acc_f[:NN], acc_f[ACC_ROWS:ACC_ROWS + NN], den_col, x0,
                     lp['bias'].reshape(1, HID), gscale,
                     lp['beta'].reshape(1, HID),
                     lp['w_ih'].T, lp['w_hh'].T,
                     lp['b_ih'].reshape(1, 3 * HID),
                     lp['b_hh'].reshape(1, 3 * HID))

    mp = params['mol']
    xl_mol, out = _k_molpre(x0, mp['lin_l'], batf)
    mol_gscale = (params['mol_gamma'] / jnp.sqrt(1.0 + BN_EPS)).reshape(1, HID)
    for _ in range(2):
        out = _k_mol(xl_mol, batf, out, mp['lin_r'], mp['att'].reshape(HID, 1),
                     mp['bias'].reshape(1, HID), mol_gscale,
                     params['mol_beta'].reshape(1, HID),
                     params['mol_w_ih'].T, params['mol_w_hh'].T,
                     params['mol_b_ih'].reshape(1, 3 * HID),
                     params['mol_b_hh'].reshape(1, 3 * HID))
    return _k_final(out, params['w_out'], params['b_out'].reshape(1, OUTD))


# speedup vs baseline: 3.2953x; 1.0207x over previous
import jax
import jax.numpy as jnp
from jax import lax
from jax.experimental import pallas as pl
from jax.experimental.pallas import tpu as pltpu
from jax.experimental.pallas import tpu_sc as plsc

NN = 10000
EE = 160000
HID = 256
NG = 64
OUTD = 128
BN_EPS = 1e-5
NS_GAT = 0.2
NS = 0.01

NSC = 2
NSUB = 16
LANES = 16
NWORK = NSC * NSUB

EP = 163840
EPW1 = EP // NWORK
C1 = 64
NCH1 = EPW1 // C1
EPW2 = EP // NSUB
C2 = 64
NCH2 = EPW2 // C2
RW = 128
DEN_ROWS = 80
DEN_BASE = 10016
ACC_ROWS = 10112
F32 = jnp.float32


def _lrelu(v, s):
    return jnp.maximum(v, v * s)


def _lane_shuffle(v, idx):
    dn = lax.GatherDimensionNumbers(offset_dims=(), collapsed_slice_dims=(0,),
                                    start_index_map=(0,))
    return lax.gather(v, idx[:, None], dn, slice_sizes=(1,),
                      mode=lax.GatherScatterMode.PROMISE_IN_BOUNDS)


def _x0_body(x_ref, w_ref, b_ref, o_ref):
    o_ref[...] = jnp.dot(x_ref[...], w_ref[...],
                         preferred_element_type=F32) + b_ref[...]


def _k_x0(x, w, b):
    return pl.pallas_call(
        _x0_body, grid=(10,),
        in_specs=[pl.BlockSpec((1000, 128), lambda i: (i, 0)),
                  pl.BlockSpec((128, HID), lambda i: (0, 0)),
                  pl.BlockSpec((1, HID), lambda i: (0, 0))],
        out_specs=pl.BlockSpec((1000, HID), lambda i: (i, 0)),
        out_shape=jax.ShapeDtypeStruct((NN, HID), F32),
    )(x, w, b)


def _el_body(ea_ref, w0, w1, w2, b0, b1, b2, o0, o1, o2):
    ea = ea_ref[...]
    o0[...] = jnp.dot(ea, w0[...], preferred_element_type=F32) + b0[...]
    o1[...] = jnp.dot(ea, w1[...], preferred_element_type=F32) + b1[...]
    o2[...] = jnp.dot(ea, w2[...], preferred_element_type=F32) + b2[...]


def _k_el(ea_p, ws, bs):
    wspec = pl.BlockSpec((16, HID), lambda i: (0, 0))
    bspec = pl.BlockSpec((1, HID), lambda i: (0, 0))
    espec = pl.BlockSpec((2048, HID), lambda i: (i, 0))
    return pl.pallas_call(
        _el_body, grid=(EP // 2048,),
        in_specs=[pl.BlockSpec((2048, 16), lambda i: (i, 0)),
                  wspec, wspec, wspec, bspec, bspec, bspec],
        out_specs=[espec, espec, espec],
        out_shape=[jax.ShapeDtypeStruct((EP, HID), F32)] * 3,
    )(ea_p, ws[0], ws[1], ws[2], bs[0], bs[1], bs[2])


def _pre_body(x_ref, wl, wr, oxf, oxl, oxr):
    x0 = x_ref[...]
    xl = jnp.dot(x0, wl[...], preferred_element_type=F32)
    oxf[...] = xl
    oxl[0, :, :] = xl[:, :128]
    oxl[1, :, :] = xl[:, 128:]
    oxr[...] = jnp.dot(x0, wr[...], preferred_element_type=F32)


def _k_pre(x0, lin_l, lin_r):
    return pl.pallas_call(
        _pre_body, grid=(10,),
        in_specs=[pl.BlockSpec((1000, HID), lambda i: (i, 0)),
                  pl.BlockSpec((HID, HID), lambda i: (0, 0)),
                  pl.BlockSpec((HID, HID), lambda i: (0, 0))],
        out_specs=[pl.BlockSpec((1000, HID), lambda i: (i, 0)),
                   pl.BlockSpec((2, 1000, 128), lambda i: (0, i, 0)),
                   pl.BlockSpec((1000, HID), lambda i: (i, 0))],
        out_shape=[jax.ShapeDtypeStruct((NN, HID), F32),
                   jax.ShapeDtypeStruct((2, NN, 128), F32),
                   jax.ShapeDtypeStruct((NN, HID), F32)],
    )(x0, lin_l, lin_r)


def _post_body(alo, ahi, dena_ref, denb_ref, x_ref, bias, gamma, beta,
               wih, whh, bih, bhh, o_ref):
    num = jnp.concatenate([alo[...], ahi[...]], axis=1)
    den = dena_ref[...] + denb_ref[...]
    gat = jnp.where(den > 0, num / den, 0.0) + bias[...]
    h = gat * gamma[...] + beta[...]
    h = jnp.where(h > 0, h, jnp.exp(h) - 1.0)
    xo = x_ref[...]
    gi = jnp.dot(h, wih[...], preferred_element_type=F32) + bih[...]
    gh = jnp.dot(xo, whh[...], preferred_element_type=F32) + bhh[...]
    r = jax.nn.sigmoid(gi[:, :HID] + gh[:, :HID])
    z = jax.nn.sigmoid(gi[:, HID:2 * HID] + gh[:, HID:2 * HID])
    n = jnp.tanh(gi[:, 2 * HID:] + r * gh[:, 2 * HID:])
    o_ref[...] = _lrelu((1.0 - z) * n + z * xo, NS)


def _k_post(acc_lo, acc_hi, den_a, den_b, x0, bias, gammas, beta,
            wihT, whhT, bih, bhh):
    vspec = pl.BlockSpec((1, HID), lambda i: (0, 0))
    gspec = pl.BlockSpec((1, 3 * HID), lambda i: (0, 0))
    return pl.pallas_call(
        _post_body, grid=(10,),
        in_specs=[pl.BlockSpec((1000, 128), lambda i: (i, 0)),
                  pl.BlockSpec((1000, 128), lambda i: (i, 0)),
                  pl.BlockSpec((1000, 1), lambda i: (i, 0)),
                  pl.BlockSpec((1000, 1), lambda i: (i, 0)),
                  pl.BlockSpec((1000, HID), lambda i: (i, 0)),
                  vspec, vspec, vspec,
                  pl.BlockSpec((HID, 3 * HID), lambda i: (0, 0)),
                  pl.BlockSpec((HID, 3 * HID), lambda i: (0, 0)),
                  gspec, gspec],
        out_specs=pl.BlockSpec((1000, HID), lambda i: (i, 0)),
        out_shape=jax.ShapeDtypeStruct((NN, HID), F32),
    )(acc_lo, acc_hi, den_a, den_b, x0, bias, gammas, beta,
      wihT, whhT, bih, bhh)


def _molpre_body(x_ref, w_ref, bat_ref, oxl, oout):
    x3 = x_ref[...]
    oxl[...] = jnp.dot(x3, w_ref[...], preferred_element_type=F32)
    gid = lax.broadcasted_iota(jnp.int32, (NG, NN), 0)
    oh = (gid == jnp.reshape(bat_ref[...], (1, NN))).astype(F32)
    oout[...] = _lrelu(jnp.dot(oh, x3, preferred_element_type=F32), NS)


def _k_molpre(x3, lin_l, batf):
    return pl.pallas_call(
        _molpre_body,
        out_shape=[jax.ShapeDtypeStruct((NN, HID), F32),
                   jax.ShapeDtypeStruct((NG, HID), F32)],
    )(x3, lin_l, batf)


def _mol_body(xl_ref, bat_ref, op_ref, wr, att, bias, gamma, beta,
              wih, whh, bih, bhh, o_ref):
    xl = xl_ref[...]
    outp = op_ref[...]
    xr = jnp.dot(outp, wr[...], preferred_element_type=F32)
    batf = bat_ref[...]
    ohT = (batf == lax.broadcasted_iota(jnp.int32, (NN, NG), 1)).astype(F32)
    xr_exp = jnp.dot(ohT, xr, preferred_element_type=F32)
    m = _lrelu(xl + xr_exp, NS_GAT)
    ex = jnp.exp(jnp.dot(m, att[...], preferred_element_type=F32))
    oh = (lax.broadcasted_iota(jnp.int32, (NG, NN), 0)
          == jnp.reshape(batf, (1, NN))).astype(F32)
    num = jnp.dot(oh, xl * ex, preferred_element_type=F32)
    den = jnp.dot(oh, ex, preferred_element_type=F32)
    gat = jnp.where(den > 0, num / den, 0.0) + bias[...]
    h = gat * gamma[...] + beta[...]
    h = jnp.where(h > 0, h, jnp.exp(h) - 1.0)
    gi = jnp.dot(h, wih[...], preferred_element_type=F32) + bih[...]
    gh = jnp.dot(outp, whh[...], preferred_element_type=F32) + bhh[...]
    r = jax.nn.sigmoid(gi[:, :HID] + gh[:, :HID])
    z = jax.nn.sigmoid(gi[:, HID:2 * HID] + gh[:, HID:2 * HID])
    n = jnp.tanh(gi[:, 2 * HID:] + r * gh[:, 2 * HID:])
    o_ref[...] = _lrelu((1.0 - z) * n + z * outp, NS)


def _k_mol(xl_mol, batf, outp, wr, att, bias, gamma, beta, wih, whh, bih, bhh):
    return pl.pallas_call(
        _mol_body,
        out_shape=jax.ShapeDtypeStruct((NG, HID), F32),
    )(xl_mol, batf, outp, wr, att, bias, gamma, beta, wih, whh, bih, bhh)


def _final_body(o_ref, w_ref, b_ref, out_ref):
    out_ref[...] = jnp.dot(o_ref[...], w_ref[...],
                           preferred_element_type=F32) + b_ref[...]


def _k_final(out, w, b):
    return pl.pallas_call(
        _final_body,
        out_shape=jax.ShapeDtypeStruct((NG, OUTD), F32),
    )(out, w, b)


_MESH = plsc.VectorSubcoreMesh(core_axis_name="c", subcore_axis_name="s",
                               num_cores=NSC, num_subcores=NSUB)


_SC_PARAMS = pltpu.CompilerParams(needs_layout_passes=False)


def _sc_pass1_body(xl_full, xr, el, srcp, dstg, atts, ex_out,
                   att_v, src_w, dst_w, xl_b, xr_b, el_b, ex_rep, ex_b,
                   sem0, sem1):
    c = lax.axis_index("c")
    s = lax.axis_index("s")
    wid = s * NSC + c
    ebase = wid * EPW1
    pltpu.sync_copy(atts, att_v)
    pltpu.sync_copy(srcp.at[pl.ds(ebase, EPW1)], src_w)
    pltpu.sync_copy(dstg.at[pl.ds(ebase, EPW1)], dst_w)
    iota16 = lax.broadcasted_iota(jnp.int32, (LANES,), 0)
    sems = (sem0, sem1)
    att_regs = [att_v[pl.ds(k * LANES, LANES)] for k in range(16)]

    def issue(g, b):
        base = g * C1
        pltpu.async_copy(xl_full.at[src_w.at[pl.ds(base, C1)]],
                         xl_b.at[b], sems[b])
        pltpu.async_copy(xr.at[dst_w.at[pl.ds(base, C1)]],
                         xr_b.at[b], sems[b])
        pltpu.async_copy(el.at[pl.ds(ebase + base, C1)], el_b.at[b], sems[b])

    def drain(g, b):
        base = g * C1
        pltpu.make_async_copy(xl_full.at[src_w.at[pl.ds(base, C1)]],
                              xl_b.at[b], sems[b]).wait()
        pltpu.make_async_copy(xr.at[dst_w.at[pl.ds(base, C1)]],
                              xr_b.at[b], sems[b]).wait()
        pltpu.make_async_copy(el.at[pl.ds(ebase + base, C1)],
                              el_b.at[b], sems[b]).wait()

    def compute(g, b):
        def edge_body(i, carry2):
            acc = jnp.zeros((LANES,), F32)
            for k in range(16):
                sl = pl.ds(k * LANES, LANES)
                m = xl_b[b, i, sl] + xr_b[b, i, sl] + el_b[b, i, sl]
                m = jnp.maximum(m, m * NS_GAT)
                acc = acc + m * att_regs[k]
            for sh in (1, 2, 4, 8):
                acc = acc + _lane_shuffle(acc, iota16 ^ sh)
            ex_rep[i, :] = jnp.exp(acc)
            return carry2

        lax.fori_loop(0, C1, edge_body, 0)
        for gg in range(C1 // LANES):
            dg = plsc.load_gather(ex_rep, [gg * LANES + iota16, iota16])
            ex_b[pl.ds(gg * LANES, LANES)] = dg
        pltpu.sync_copy(ex_b, ex_out.at[pl.ds(ebase + g * C1, C1)])

    issue(0, 0)
    issue(1, 1)

    def pair_body(p, carry):
        for b in (0, 1):
            g = 2 * p + b
            drain(g, b)
            compute(g, b)
            issue(g + 2, b)
        return carry

    lax.fori_loop(0, (NCH1 - 2) // 2, pair_body, 0)
    for b in (0, 1):
        drain(NCH1 - 2 + b, b)
        compute(NCH1 - 2 + b, b)


def _sc_pass1(xl_full, xr, el, src_p, dstg_p, att):
    return pl.kernel(
        _sc_pass1_body,
        out_type=jax.ShapeDtypeStruct((EP,), F32),
        mesh=_MESH,
        compiler_params=_SC_PARAMS,
        scratch_types=[
            pltpu.VMEM((HID,), F32),
            pltpu.VMEM((EPW1,), jnp.int32),
            pltpu.VMEM((EPW1,), jnp.int32),
            pltpu.VMEM((2, C1, HID), F32),
            pltpu.VMEM((2, C1, HID), F32),
            pltpu.VMEM((2, C1, HID), F32),
            pltpu.VMEM((C1, LANES), F32),
            pltpu.VMEM((C1,), F32),
            pltpu.SemaphoreType.DMA,
            pltpu.SemaphoreType.DMA,
        ],
    )(xl_full, xr, el, src_p, dstg_p, att)


def _sc_pass2_body(xl_cat, srcp, dsts, ex, accf_out, den_out,
                   src_b, dst_b, ex_b, xl_b, contrib, contrib_d,
                   dstS, drowS, dcolS, spacc_f, gsem0, gsem1, ssem0, ssem1):
    c = lax.axis_index("c")
    s = lax.axis_index("s")
    ebase = s * EPW2
    iota16 = lax.broadcasted_iota(jnp.int32, (LANES,), 0)
    zero16 = jnp.zeros((LANES,), F32)
    zero16i = jnp.zeros((LANES,), jnp.int32)
    gsems = (gsem0, gsem1)
    ssems = (ssem0, ssem1)
    cNN = c * NN

    def zrow(r, carry):
        for b in (0, 1):
            for kk in range(128 // LANES):
                sl = pl.ds(kk * LANES, LANES)
                contrib[b, r, sl] = zero16
        return carry
    lax.fori_loop(0, C2, zrow, 0)

    def zrowd(r, carry):
        for b in (0, 1):
            for kk in range(128 // LANES):
                sl = pl.ds(kk * LANES, LANES)
                contrib_d[b, r, sl] = zero16
        return carry
    lax.fori_loop(0, 32, zrowd, 0)

    for b in (0, 1):
        for g2 in range(C2 // LANES):
            dstS[b, pl.ds(g2 * LANES, LANES)] = zero16i
        for g2 in range(2):
            drowS[b, pl.ds(g2 * LANES, LANES)] = zero16i
            dcolS[b, pl.ds(g2 * LANES, LANES)] = zero16i

    def zacc(z, carry):
        pltpu.sync_copy(contrib.at[0], spacc_f.at[pl.ds(s * 632 + z * C2, C2)])
        return carry
    lax.fori_loop(0, 9, zacc, 0)
    pltpu.sync_copy(contrib.at[0].at[pl.ds(0, 56)],
                    spacc_f.at[pl.ds(s * 632 + 576, 56)])
    plsc.subcore_barrier()

    def issue_scatter(b):
        pltpu.async_copy(contrib.at[b], spacc_f.at[dstS.at[b]],
                         ssems[b], add=True)
        pltpu.async_copy(contrib_d.at[b], spacc_f.at[drowS.at[b]],
                         ssems[b], add=True)

    def wait_scatter(b):
        pltpu.make_async_copy(contrib.at[b], spacc_f.at[dstS.at[b]],
                              ssems[b]).wait()
        pltpu.make_async_copy(contrib_d.at[b], spacc_f.at[drowS.at[b]],
                              ssems[b]).wait()

    def issue_gather(g, b):
        base = ebase + g * C2
        pltpu.sync_copy(srcp.at[pl.ds(base, C2)], src_b.at[b])
        pltpu.sync_copy(dsts.at[pl.ds(base, C2)], dst_b.at[b])
        pltpu.sync_copy(ex.at[pl.ds(base, C2)], ex_b.at[b])
        for g2 in range(C2 // LANES):
            sl = pl.ds(g2 * LANES, LANES)
            src_b[b, sl] = src_b[b, sl] + cNN
        pltpu.async_copy(xl_cat.at[src_b.at[b]], xl_b.at[b], gsems[b])

    def drain_gather(b):
        pltpu.make_async_copy(xl_cat.at[src_b.at[b]], xl_b.at[b],
                              gsems[b]).wait()

    def zero_cells(b):
        bvec = jnp.full((LANES,), b, jnp.int32)
        for g2 in range(2):
            rowsg = g2 * LANES + iota16
            dcolv = dcolS[b, pl.ds(g2 * LANES, LANES)]
            plsc.store_scatter(contrib_d, [bvec, rowsg, dcolv], zero16)

    def compute(g, b):
        bvec = jnp.full((LANES,), b, jnp.int32)

        def edge_body(i, cr):
            exg = plsc.load_gather(ex_b, [bvec, jnp.full((LANES,), 0,
                                                         jnp.int32) + i])
            for k in range(8):
                sl = pl.ds(k * LANES, LANES)
                contrib[b, i, sl] = xl_b[b, i, sl] * exg
            return cr
        lax.fori_loop(0, C2, edge_body, 0)

        for g2 in range(C2 // LANES):
            sl = pl.ds(g2 * LANES, LANES)
            dstS[b, sl] = dst_b[b, sl]
        for g2 in range(2):
            off = c * 32 + g2 * LANES
            dstv = dst_b[b, pl.ds(off, LANES)]
            drowS[b, pl.ds(g2 * LANES, LANES)] = DEN_BASE + \
                jnp.right_shift(dstv, 7)
            dcol = dstv & 127
            dcolS[b, pl.ds(g2 * LANES, LANES)] = dcol
            exg2 = plsc.load_gather(ex_b, [bvec, off + iota16])
            plsc.store_scatter(contrib_d, [bvec, g2 * LANES + iota16, dcol],
                               exg2)

    for b in (0, 1):
        issue_scatter(b)
        issue_gather(b, b)

    def pair_body(p, carry):
        for b in (0, 1):
            g = 2 * p + b
            drain_gather(b)
            wait_scatter(b)
            zero_cells(b)
            compute(g, b)
            issue_scatter(b)
            issue_gather(g + 2, b)
        return carry

    lax.fori_loop(0, (NCH2 - 2) // 2, pair_body, 0)
    for b in (0, 1):
        g = NCH2 - 2 + b
        drain_gather(b)
        wait_scatter(b)
        zero_cells(b)
        compute(g, b)
        issue_scatter(b)
    for b in (0, 1):
        wait_scatter(b)

    plsc.subcore_barrier()

    pltpu.sync_copy(spacc_f.at[pl.ds(s * 632, 632)],
                    accf_out.at[pl.ds(c * ACC_ROWS + s * 632, 632)])

    @pl.when(s == 0)
    def _():
        pltpu.sync_copy(spacc_f.at[pl.ds(DEN_BASE, DEN_ROWS)],
                        den_out.at[pl.ds(c * DEN_ROWS, DEN_ROWS)])


def _sc_pass2(xl_cat, src_p, dsts_p, ex):
    return pl.kernel(
        _sc_pass2_body,
        out_type=(jax.ShapeDtypeStruct((2 * ACC_ROWS, 128), F32),
                  jax.ShapeDtypeStruct((2 * DEN_ROWS, 128), F32)),
        mesh=_MESH,
        compiler_params=_SC_PARAMS,
        scratch_types=[
            pltpu.VMEM((2, C2), jnp.int32),
            pltpu.VMEM((2, C2), jnp.int32),
            pltpu.VMEM((2, C2), F32),
            pltpu.VMEM((2, C2, 128), F32),
            pltpu.VMEM((2, C2, 128), F32),
            pltpu.VMEM((2, 32, 128), F32),
            pltpu.VMEM((2, C2), jnp.int32),
            pltpu.VMEM((2, 32), jnp.int32),
            pltpu.VMEM((2, 32), jnp.int32),
            pltpu.VMEM_SHARED((ACC_ROWS, 128), F32),
            pltpu.SemaphoreType.DMA,
            pltpu.SemaphoreType.DMA,
            pltpu.SemaphoreType.DMA,
            pltpu.SemaphoreType.DMA,
        ],
    )(xl_cat, src_p, dsts_p, ex)


def kernel(x, edge_attr, params, edge_index, batch):
    src, dst = edge_index[0], edge_index[1]
    pad = EP - EE
    src_p = jnp.pad(src, (0, pad))
    dstg_p = jnp.pad(dst, (0, pad))
    dsts_p = jnp.pad(dst, (0, pad), constant_values=NN)
    ea_p = jnp.pad(edge_attr, ((0, pad), (0, 0)))
    batf = batch.reshape(NN, 1)

    we = [params['w_edge'] @ lp['lin_edge'] for lp in params['layers']]
    be = [(params['b_edge'] @ lp['lin_edge']).reshape(1, HID)
          for lp in params['layers']]

    x0 = _k_x0(x, params['w_node'], params['b_node'].reshape(1, HID))
    els = _k_el(ea_p, we, be)

    for li, lp in enumerate(params['layers']):
        xl_full, xl2, xr = _k_pre(x0, lp['lin_l'], lp['lin_r'])
        xl_cat = xl2.reshape(2 * NN, 128)
        ex = _sc_pass1(xl_full, xr, els[li], src_p, dstg_p, lp['att'])
        acc_f, den = _sc_pass2(xl_cat, src_p, dsts_p, ex)
        den_a = den[:DEN_ROWS].reshape(DEN_ROWS * 128)[:NN].reshape(NN, 1)
        den_b = den[DEN_ROWS:].reshape(DEN_ROWS * 128)[:NN].reshape(NN, 1)
        gscale = (lp['gamma'] / jnp.sqrt(1.0 + BN_EPS)).reshape(1, HID)
        x0 = _k_post(acc_f[:NN], acc_f[ACC_ROWS:ACC_ROWS + NN], den_a, den_b, x0,
                     lp['bias'].reshape(1, HID), gscale,
                     lp['beta'].reshape(1, HID),
                     lp['w_ih'].T, lp['w_hh'].T,
                     lp['b_ih'].reshape(1, 3 * HID),
                     lp['b_hh'].reshape(1, 3 * HID))

    mp = params['mol']
    xl_mol, out = _k_molpre(x0, mp['lin_l'], batf)
    mol_gscale = (params['mol_gamma'] / jnp.sqrt(1.0 + BN_EPS)).reshape(1, HID)
    for _ in range(2):
        out = _k_mol(xl_mol, batf, out, mp['lin_r'], mp['att'].reshape(HID, 1),
                     mp['bias'].reshape(1, HID), mol_gscale,
                     params['mol_beta'].reshape(1, HID),
                     params['mol_w_ih'].T, params['mol_w_hh'].T,
                     params['mol_b_ih'].reshape(1, 3 * HID),
                     params['mol_b_hh'].reshape(1, 3 * HID))
    return _k_final(out, params['w_out'], params['b_out'].reshape(1, OUTD))
